# baseline (speedup 1.0000x reference)
import jax
import jax.numpy as jnp
from jax import lax
from jax.experimental import pallas as pl
from jax.experimental.pallas import tpu as pltpu
from jax.experimental.pallas import tpu_sc as plsc

N = 50000
E = 800000
FIN = 100
FINP = 112
HID = 256
FOUT = 47
FOUTP = 48

NC = 2
NS = 16
NW = NC * NS
RB = E // 128
SBR = 16
NSB = 416
SBW = NSB // NW
RBP = NSB * SBR
NP = 51200
RPT = NP // NS
ZR = 400
ZI = RPT // ZR
R = 512
GRID = NP // R
GRID3 = (N + R - 1) // R
PV = NP * 16 // 128
RV = R * 16 // 128


def _make_segsum(K, C, ndeg):
    KT = K + ndeg
    mesh = plsc.VectorSubcoreMesh(core_axis_name="c", subcore_axis_name="s")

    def body(src_ref, dst_ref, *rest):
        tables = rest[:K]
        out_ref = rest[K]
        idx_v, dst_v, rows_v, zbuf, acc, gsem = rest[K + 1:]
        cid = lax.axis_index("c")
        sid = lax.axis_index("s")
        wid = sid * NC + cid
        tbase = sid * RPT
        sb0 = wid * SBW

        zv = jnp.zeros((16,), jnp.float32)

        def zrow(i, carry):
            for j in range(C // 16):
                zbuf[i, pl.ds(j * 16, 16)] = zv
            return carry

        lax.fori_loop(0, ZR, zrow, 0)

        def zero_stripe():
            for j in range(ZI):
                pltpu.sync_copy(zbuf, acc.at[pl.ds(tbase + j * ZR, ZR)])

        def writeout(slot, rezero):
            for j in range(ZI):
                sl = pl.ds(tbase + j * ZR, ZR)
                off = (cid * KT + slot) * NP + tbase + j * ZR
                pltpu.sync_copy(acc.at[sl], out_ref.at[pl.ds(off, ZR)])
                if rezero:
                    pltpu.sync_copy(zbuf, acc.at[sl])

        zero_stripe()

        if ndeg:
            ov = jnp.ones((16,), jnp.float32)

            def orow(i, carry):
                for j in range(C // 16):
                    rows_v[0, i, pl.ds(j * 16, 16)] = ov
                return carry

            lax.fori_loop(0, 128, orow, 0)
            plsc.subcore_barrier()

            def dbody(t, carry):
                row0 = (sb0 + t) * SBR
                pltpu.sync_copy(dst_ref.at[pl.ds(row0, SBR)], dst_v)
                for k in range(SBR):
                    @pl.when(row0 + k < RB)
                    def _():
                        pltpu.sync_copy(rows_v.at[0],
                                        acc.at[dst_v.at[k]], add=True)
                return carry

            lax.fori_loop(0, SBW, dbody, 0)
            plsc.subcore_barrier()
            writeout(K, True)

        for c in range(K):
            tslice = tables[c]

            def cbody(t, carry, tslice=tslice):
                row0 = (sb0 + t) * SBR
                pltpu.sync_copy(src_ref.at[pl.ds(row0, SBR)], idx_v)
                descs = [
                    pltpu.async_copy(tslice.at[idx_v.at[k]], rows_v.at[k],
                                     gsem)
                    for k in range(SBR)
                ]
                pltpu.sync_copy(dst_ref.at[pl.ds(row0, SBR)], dst_v)
                for d in descs:
                    d.wait()
                for k in range(SBR):
                    @pl.when(row0 + k < RB)
                    def _():
                        pltpu.sync_copy(rows_v.at[k], acc.at[dst_v.at[k]],
                                        add=True)
                return carry

            plsc.subcore_barrier()
            lax.fori_loop(0, SBW, cbody, 0)
            plsc.subcore_barrier()
            writeout(c, c < K - 1)

    out_type = jax.ShapeDtypeStruct((2 * KT * NP, C), jnp.float32)
    scratch = [
        pltpu.VMEM((SBR, 128), jnp.int32),
        pltpu.VMEM((SBR, 128), jnp.int32),
        pltpu.VMEM((SBR, 128, C), jnp.float32),
        pltpu.VMEM((ZR, C), jnp.float32),
        pltpu.VMEM_SHARED((NP, C), jnp.float32),
        pltpu.SemaphoreType.DMA,
    ]
    return pl.kernel(body, out_type=out_type, mesh=mesh,
                     scratch_types=scratch,
                     compiler_params=pltpu.CompilerParams(
                         use_tc_tiling_on_sc=False))


_seg1 = _make_segsum(7, 16, 1)
_seg2 = _make_segsum(16, 16, 0)
_seg3 = _make_segsum(3, 16, 0)


def _pack(h):
    return jnp.concatenate([h[64 * k:64 * k + 64, :] for k in range(8)],
                           axis=1)


def _unpack_a(v):
    return jnp.concatenate([v[:, 16 * k:16 * k + 16] for k in range(8)],
                           axis=0)


def _unpack(ref):
    return _unpack_a(ref[...])


def _chunker_body(x_ref, *outs):
    x = x_ref[...]
    pad = jnp.zeros((R, FINP - FIN), jnp.float32)
    xp = jnp.concatenate([x, pad], axis=1)
    for c in range(7):
        outs[c][...] = _pack(xp[:, 16 * c:16 * c + 16])


_chunk_x = pl.pallas_call(
    _chunker_body,
    grid=(GRID,),
    in_specs=[pl.BlockSpec((R, FIN), lambda i: (i, 0))],
    out_specs=[pl.BlockSpec((RV, 128), lambda i: (i, 0))] * 7,
    out_shape=[jax.ShapeDtypeStruct((PV, 128), jnp.float32)] * 7,
)


def _agg_specs(KT, slots):
    specs = []
    for g in range(2):
        for s in slots:
            base = (g * KT + s) * (PV // RV)
            specs.append(pl.BlockSpec(
                (RV, 128), lambda i, base=base: (base + i, 0)))
    return specs


def _deg_recip(d0, d1):
    deg = (_unpack(d0) + _unpack(d1))[:, 0:1]
    return 1.0 / jnp.maximum(deg, 1.0)


def _l1_body(x_ref, *refs):
    aggs = refs[:16]
    ws_ref, wn_ref, b_ref = refs[16:19]
    outs = refs[19:]
    recip = _deg_recip(aggs[7], aggs[15])
    a = jnp.concatenate(
        [_unpack(aggs[c]) + _unpack(aggs[8 + c]) for c in range(7)], axis=1)
    a = a * recip
    h = jnp.dot(x_ref[...], ws_ref[...], preferred_element_type=jnp.float32)
    h = h + jnp.dot(a, wn_ref[...], preferred_element_type=jnp.float32)
    h = jnp.maximum(h + b_ref[...], 0.0)
    for c in range(16):
        outs[c][...] = _pack(h[:, 16 * c:16 * c + 16])


_l1 = pl.pallas_call(
    _l1_body,
    grid=(GRID,),
    in_specs=(
        [pl.BlockSpec((R, FIN), lambda i: (i, 0))]
        + _agg_specs(8, range(8))
        + [
            pl.BlockSpec((FIN, HID), lambda i: (0, 0)),
            pl.BlockSpec((FINP, HID), lambda i: (0, 0)),
            pl.BlockSpec((1, HID), lambda i: (0, 0)),
        ]
    ),
    out_specs=[pl.BlockSpec((RV, 128), lambda i: (i, 0))] * 16,
    out_shape=[jax.ShapeDtypeStruct((PV, 128), jnp.float32)] * 16,
)


def _l2_body(*refs):
    hrefs = refs[:16]
    aggs = refs[16:48]
    d0, d1 = refs[48:50]
    ws_ref, wn_ref, b_ref, wn3_ref, h2_out = refs[50:55]
    mouts = refs[55:]
    h = jnp.concatenate([_unpack(r) for r in hrefs], axis=1)
    recip = _deg_recip(d0, d1)
    a = jnp.concatenate(
        [_unpack(aggs[c]) + _unpack(aggs[16 + c]) for c in range(16)],
        axis=1)
    a = a * recip
    hh = jnp.dot(h, ws_ref[...], preferred_element_type=jnp.float32)
    hh = hh + jnp.dot(a, wn_ref[...], preferred_element_type=jnp.float32)
    hh = jnp.maximum(hh + b_ref[...], 0.0)
    h2_out[...] = hh
    m3 = jnp.dot(hh, wn3_ref[...], preferred_element_type=jnp.float32)
    for c in range(3):
        mouts[c][...] = _pack(m3[:, 16 * c:16 * c + 16])


_l2 = pl.pallas_call(
    _l2_body,
    grid=(GRID,),
    in_specs=(
        [pl.BlockSpec((RV, 128), lambda i: (i, 0))] * 16
        + _agg_specs(16, range(16))
        + _agg_specs(8, [7])
        + [
            pl.BlockSpec((HID, HID), lambda i: (0, 0)),
            pl.BlockSpec((HID, HID), lambda i: (0, 0)),
            pl.BlockSpec((1, HID), lambda i: (0, 0)),
            pl.BlockSpec((HID, FOUTP), lambda i: (0, 0)),
        ]
    ),
    out_specs=[pl.BlockSpec((R, HID), lambda i: (i, 0))]
    + [pl.BlockSpec((RV, 128), lambda i: (i, 0))] * 3,
    out_shape=[jax.ShapeDtypeStruct((NP, HID), jnp.float32)]
    + [jax.ShapeDtypeStruct((PV, 128), jnp.float32)] * 3,
)


def _l3_body(h_ref, *refs):
    aggs = refs[:6]
    d0, d1 = refs[6:8]
    ws_ref, b_ref, out_ref = refs[8:]
    recip = _deg_recip(d0, d1)
    a = jnp.concatenate(
        [_unpack(aggs[c]) + _unpack(aggs[3 + c]) for c in range(3)], axis=1)
    a = a * recip
    s = jnp.dot(h_ref[...], ws_ref[...], preferred_element_type=jnp.float32)
    s = s + a + b_ref[...]
    col = lax.broadcasted_iota(jnp.int32, (R, FOUTP), 1)
    valid = col < FOUT
    s = jnp.where(valid, s, -jnp.inf)
    m = jnp.max(s, axis=1, keepdims=True)
    e = jnp.where(valid, jnp.exp(s - m), 0.0)
    lse = jnp.log(jnp.sum(e, axis=1, keepdims=True))
    r = s - (m + lse)
    out_ref[...] = r[:, :FOUT]


_l3 = pl.pallas_call(
    _l3_body,
    grid=(GRID3,),
    in_specs=(
        [pl.BlockSpec((R, HID), lambda i: (i, 0))]
        + _agg_specs(3, range(3))
        + _agg_specs(8, [7])
        + [
            pl.BlockSpec((HID, FOUTP), lambda i: (0, 0)),
            pl.BlockSpec((1, FOUTP), lambda i: (0, 0)),
        ]
    ),
    out_specs=pl.BlockSpec((R, FOUT), lambda i: (i, 0)),
    out_shape=jax.ShapeDtypeStruct((N, FOUT), jnp.float32),
)


def _as_tables(view_arrays):
    return [v.reshape(NP, 16) for v in view_arrays]


def kernel(x, local_edges_list, remote_edges_list, W_self1, W_neigh1, b1,
           W_self2, W_neigh2, b2, W_self3, W_neigh3, b3):
    ei = local_edges_list
    eb, ej = ei // 512, ei % 512
    eip = eb * 512 + (ej % 64) * 8 + ej // 64
    src2 = jnp.pad(eip[0].reshape(RB, 128), ((0, RBP - RB), (0, 0)))
    dst2 = jnp.pad(eip[1].reshape(RB, 128), ((0, RBP - RB), (0, 0)))
    wn1p = jnp.pad(W_neigh1, ((0, FINP - FIN), (0, 0)))
    wn3p = jnp.pad(W_neigh3, ((0, 0), (0, FOUTP - FOUT)))
    ws3p = jnp.pad(W_self3, ((0, 0), (0, FOUTP - FOUT)))
    b3p = jnp.pad(b3, (0, FOUTP - FOUT)).reshape(1, FOUTP)

    xp2 = jnp.pad(x, ((0, NP - N), (0, 0)))
    xc = _chunk_x(xp2)
    agg1 = _seg1(src2, dst2, *_as_tables(xc)).reshape(2 * 8 * PV, 128)
    h1c = _l1(xp2, *([agg1] * 16), W_self1, wn1p, b1.reshape(1, HID))
    agg2 = _seg2(src2, dst2, *_as_tables(h1c)).reshape(2 * 16 * PV, 128)
    h2, m0, m1, m2 = _l2(*h1c, *([agg2] * 32), agg1, agg1, W_self2,
                         W_neigh2, b2.reshape(1, HID), wn3p)
    agg3 = _seg3(src2, dst2, *_as_tables((m0, m1, m2))).reshape(
        2 * 3 * PV, 128)
    return _l3(h2, *([agg3] * 6), agg1, agg1, ws3p, b3p)


# speedup vs baseline: 2.3827x; 2.3827x over previous
import jax
import jax.numpy as jnp
from jax import lax
from jax.experimental import pallas as pl
from jax.experimental.pallas import tpu as pltpu
from jax.experimental.pallas import tpu_sc as plsc

N = 50000
E = 800000
FIN = 100
FINP = 112
HID = 256
FOUT = 47
FOUTP = 48

NC = 2
NS = 16
NW = NC * NS
RB = E // 128
SBR = 16
NSB = (RB + SBR - 1) // SBR
RBP = NSB * SBR
NP = 51200
RPT = NP // NS
ZR = 400
ZI = RPT // ZR
R = 512
GRID = NP // R
GRID3 = (N + R - 1) // R
PV = NP * 16 // 128
RV = R * 16 // 128


def _make_segsum(K, C, ndeg):
    KT = K + ndeg
    mesh = plsc.VectorSubcoreMesh(core_axis_name="c", subcore_axis_name="s")

    def body(src_ref, dst_ref, *rest):
        tables = rest[:K]
        out_ref = rest[K]
        idx_v, dst_v, rows_v, zbuf, acc, gsem = rest[K + 1:]
        cid = lax.axis_index("c")
        sid = lax.axis_index("s")
        wid = sid * NC + cid
        tbase = sid * RPT
        sb0 = (wid * NSB) // NW
        sb1 = ((wid + 1) * NSB) // NW

        zv = jnp.zeros((16,), jnp.float32)

        def zrow(i, carry):
            for j in range(C // 16):
                zbuf[i, pl.ds(j * 16, 16)] = zv
            return carry

        lax.fori_loop(0, ZR, zrow, 0)

        def zero_stripe():
            for j in range(ZI):
                pltpu.sync_copy(zbuf, acc.at[pl.ds(tbase + j * ZR, ZR)])

        def writeout(slot, rezero):
            for j in range(ZI):
                sl = pl.ds(tbase + j * ZR, ZR)
                off = (cid * KT + slot) * NP + tbase + j * ZR
                pltpu.sync_copy(acc.at[sl], out_ref.at[pl.ds(off, ZR)])
                if rezero:
                    pltpu.sync_copy(zbuf, acc.at[sl])

        zero_stripe()

        if ndeg:
            ov = jnp.ones((16,), jnp.float32)

            def orow(i, carry):
                for j in range(C // 16):
                    rows_v[0, i, pl.ds(j * 16, 16)] = ov
                return carry

            lax.fori_loop(0, 128, orow, 0)
            plsc.subcore_barrier()

            def dbody(t, carry):
                row0 = (sb0 + t) * SBR
                pltpu.sync_copy(dst_ref.at[pl.ds(row0, SBR)], dst_v)
                for k in range(SBR):
                    @pl.when(row0 + k < RB)
                    def _():
                        pltpu.sync_copy(rows_v.at[0],
                                        acc.at[dst_v.at[k]], add=True)
                return carry

            lax.fori_loop(0, sb1 - sb0, dbody, 0)
            plsc.subcore_barrier()
            writeout(K, True)

        for c in range(K):
            tslice = tables[c]

            def cbody(t, carry, tslice=tslice):
                row0 = (sb0 + t) * SBR
                pltpu.sync_copy(src_ref.at[pl.ds(row0, SBR)], idx_v)
                descs = [
                    pltpu.async_copy(tslice.at[idx_v.at[k]], rows_v.at[k],
                                     gsem)
                    for k in range(SBR)
                ]
                pltpu.sync_copy(dst_ref.at[pl.ds(row0, SBR)], dst_v)
                for d in descs:
                    d.wait()
                for k in range(SBR):
                    @pl.when(row0 + k < RB)
                    def _():
                        pltpu.sync_copy(rows_v.at[k], acc.at[dst_v.at[k]],
                                        add=True)
                return carry

            plsc.subcore_barrier()
            lax.fori_loop(0, sb1 - sb0, cbody, 0)
            plsc.subcore_barrier()
            writeout(c, c < K - 1)

    out_type = jax.ShapeDtypeStruct((2 * KT * NP, C), jnp.float32)
    scratch = [
        pltpu.VMEM((SBR, 128), jnp.int32),
        pltpu.VMEM((SBR, 128), jnp.int32),
        pltpu.VMEM((SBR, 128, C), jnp.float32),
        pltpu.VMEM((ZR, C), jnp.float32),
        pltpu.VMEM_SHARED((NP, C), jnp.float32),
        pltpu.SemaphoreType.DMA,
    ]
    return pl.kernel(body, out_type=out_type, mesh=mesh,
                     scratch_types=scratch,
                     compiler_params=pltpu.CompilerParams(
                         use_tc_tiling_on_sc=False))


_seg1 = _make_segsum(7, 16, 1)
_seg2 = _make_segsum(16, 16, 0)
_seg3 = _make_segsum(3, 16, 0)


def _pack(h):
    return jnp.concatenate([h[64 * k:64 * k + 64, :] for k in range(8)],
                           axis=1)


def _unpack_a(v):
    return jnp.concatenate([v[:, 16 * k:16 * k + 16] for k in range(8)],
                           axis=0)


def _unpack(ref):
    return _unpack_a(ref[...])


def _chunker_body(x_ref, *outs):
    x = x_ref[...]
    pad = jnp.zeros((R, FINP - FIN), jnp.float32)
    xp = jnp.concatenate([x, pad], axis=1)
    for c in range(7):
        outs[c][...] = _pack(xp[:, 16 * c:16 * c + 16])


_chunk_x = pl.pallas_call(
    _chunker_body,
    grid=(GRID,),
    in_specs=[pl.BlockSpec((R, FIN), lambda i: (i, 0))],
    out_specs=[pl.BlockSpec((RV, 128), lambda i: (i, 0))] * 7,
    out_shape=[jax.ShapeDtypeStruct((PV, 128), jnp.float32)] * 7,
)


def _agg_specs(KT, slots):
    specs = []
    for g in range(2):
        for s in slots:
            base = (g * KT + s) * (PV // RV)
            specs.append(pl.BlockSpec(
                (RV, 128), lambda i, base=base: (base + i, 0)))
    return specs


def _deg_recip(d0, d1):
    deg = (_unpack(d0) + _unpack(d1))[:, 0:1]
    return 1.0 / jnp.maximum(deg, 1.0)


def _l1_body(x_ref, *refs):
    aggs = refs[:16]
    ws_ref, wn_ref, b_ref = refs[16:19]
    outs = refs[19:]
    recip = _deg_recip(aggs[7], aggs[15])
    a = jnp.concatenate(
        [_unpack(aggs[c]) + _unpack(aggs[8 + c]) for c in range(7)], axis=1)
    a = a * recip
    h = jnp.dot(x_ref[...], ws_ref[...], preferred_element_type=jnp.float32)
    h = h + jnp.dot(a, wn_ref[...], preferred_element_type=jnp.float32)
    h = jnp.maximum(h + b_ref[...], 0.0)
    for c in range(16):
        outs[c][...] = _pack(h[:, 16 * c:16 * c + 16])


_l1 = pl.pallas_call(
    _l1_body,
    grid=(GRID,),
    in_specs=(
        [pl.BlockSpec((R, FIN), lambda i: (i, 0))]
        + _agg_specs(8, range(8))
        + [
            pl.BlockSpec((FIN, HID), lambda i: (0, 0)),
            pl.BlockSpec((FINP, HID), lambda i: (0, 0)),
            pl.BlockSpec((1, HID), lambda i: (0, 0)),
        ]
    ),
    out_specs=[pl.BlockSpec((RV, 128), lambda i: (i, 0))] * 16,
    out_shape=[jax.ShapeDtypeStruct((PV, 128), jnp.float32)] * 16,
)


def _l2_body(*refs):
    hrefs = refs[:16]
    aggs = refs[16:48]
    d0, d1 = refs[48:50]
    ws_ref, wn_ref, b_ref, wn3_ref, h2_out = refs[50:55]
    mouts = refs[55:]
    h = jnp.concatenate([_unpack(r) for r in hrefs], axis=1)
    recip = _deg_recip(d0, d1)
    a = jnp.concatenate(
        [_unpack(aggs[c]) + _unpack(aggs[16 + c]) for c in range(16)],
        axis=1)
    a = a * recip
    hh = jnp.dot(h, ws_ref[...], preferred_element_type=jnp.float32)
    hh = hh + jnp.dot(a, wn_ref[...], preferred_element_type=jnp.float32)
    hh = jnp.maximum(hh + b_ref[...], 0.0)
    h2_out[...] = hh
    m3 = jnp.dot(hh, wn3_ref[...], preferred_element_type=jnp.float32)
    for c in range(3):
        mouts[c][...] = _pack(m3[:, 16 * c:16 * c + 16])


_l2 = pl.pallas_call(
    _l2_body,
    grid=(GRID,),
    in_specs=(
        [pl.BlockSpec((RV, 128), lambda i: (i, 0))] * 16
        + _agg_specs(16, range(16))
        + _agg_specs(8, [7])
        + [
            pl.BlockSpec((HID, HID), lambda i: (0, 0)),
            pl.BlockSpec((HID, HID), lambda i: (0, 0)),
            pl.BlockSpec((1, HID), lambda i: (0, 0)),
            pl.BlockSpec((HID, FOUTP), lambda i: (0, 0)),
        ]
    ),
    out_specs=[pl.BlockSpec((R, HID), lambda i: (i, 0))]
    + [pl.BlockSpec((RV, 128), lambda i: (i, 0))] * 3,
    out_shape=[jax.ShapeDtypeStruct((NP, HID), jnp.float32)]
    + [jax.ShapeDtypeStruct((PV, 128), jnp.float32)] * 3,
)


def _l3_body(h_ref, *refs):
    aggs = refs[:6]
    d0, d1 = refs[6:8]
    ws_ref, b_ref, out_ref = refs[8:]
    recip = _deg_recip(d0, d1)
    a = jnp.concatenate(
        [_unpack(aggs[c]) + _unpack(aggs[3 + c]) for c in range(3)], axis=1)
    a = a * recip
    s = jnp.dot(h_ref[...], ws_ref[...], preferred_element_type=jnp.float32)
    s = s + a + b_ref[...]
    col = lax.broadcasted_iota(jnp.int32, (R, FOUTP), 1)
    valid = col < FOUT
    s = jnp.where(valid, s, -jnp.inf)
    m = jnp.max(s, axis=1, keepdims=True)
    e = jnp.where(valid, jnp.exp(s - m), 0.0)
    lse = jnp.log(jnp.sum(e, axis=1, keepdims=True))
    r = s - (m + lse)
    out_ref[...] = r[:, :FOUT]


_l3 = pl.pallas_call(
    _l3_body,
    grid=(GRID3,),
    in_specs=(
        [pl.BlockSpec((R, HID), lambda i: (i, 0))]
        + _agg_specs(3, range(3))
        + _agg_specs(8, [7])
        + [
            pl.BlockSpec((HID, FOUTP), lambda i: (0, 0)),
            pl.BlockSpec((1, FOUTP), lambda i: (0, 0)),
        ]
    ),
    out_specs=pl.BlockSpec((R, FOUT), lambda i: (i, 0)),
    out_shape=jax.ShapeDtypeStruct((N, FOUT), jnp.float32),
)


def _as_tables(view_arrays):
    return [v.reshape(NP, 16) for v in view_arrays]


def kernel(x, local_edges_list, remote_edges_list, W_self1, W_neigh1, b1,
           W_self2, W_neigh2, b2, W_self3, W_neigh3, b3):
    ei = local_edges_list
    eb, ej = ei // 512, ei % 512
    eip = eb * 512 + (ej % 64) * 8 + ej // 64
    src2 = jnp.pad(eip[0].reshape(RB, 128), ((0, RBP - RB), (0, 0)))
    dst2 = jnp.pad(eip[1].reshape(RB, 128), ((0, RBP - RB), (0, 0)))
    wn1p = jnp.pad(W_neigh1, ((0, FINP - FIN), (0, 0)))
    wn3p = jnp.pad(W_neigh3, ((0, 0), (0, FOUTP - FOUT)))
    ws3p = jnp.pad(W_self3, ((0, 0), (0, FOUTP - FOUT)))
    b3p = jnp.pad(b3, (0, FOUTP - FOUT)).reshape(1, FOUTP)

    xp2 = jnp.pad(x, ((0, NP - N), (0, 0)))
    xc = _chunk_x(xp2)
    agg1 = _seg1(src2, dst2, *_as_tables(xc)).reshape(2 * 8 * PV, 128)
    h1c = _l1(xp2, *([agg1] * 16), W_self1, wn1p, b1.reshape(1, HID))
    agg2 = _seg2(src2, dst2, *_as_tables(h1c)).reshape(2 * 16 * PV, 128)
    h2, m0, m1, m2 = _l2(*h1c, *([agg2] * 32), agg1, agg1, W_self2,
                         W_neigh2, b2.reshape(1, HID), wn3p)
    agg3 = _seg3(src2, dst2, *_as_tables((m0, m1, m2))).reshape(
        2 * 3 * PV, 128)
    return _l3(h2, *([agg3] * 6), agg1, agg1, ws3p, b3p)
